# Initial kernel scaffold; baseline (speedup 1.0000x reference)
import jax
import jax.numpy as jnp
from jax.experimental import pallas as pl


def kernel(h, edge_index, edge_feat, weight, bias):
    raise NotImplementedError("write your pallas kernel here")


# speedup vs baseline: 5.4715x; 5.4715x over previous
import functools

import jax
import jax.numpy as jnp
from jax import lax
from jax.experimental import pallas as pl
from jax.experimental.pallas import tpu as pltpu
from jax.experimental.pallas import tpu_sc as plsc

N_NODES = 10000
N_EDGES = 320000
D = 128

NC = 2
NS = 16
NW = NC * NS
CHUNK = 128
CPW = 80
EPW = CPW * CHUNK
E_PAD = NW * EPW
ROWS_PER_TILE = 624
LANES = 16


def _mm_body(h_ref, w_ref, o_ref):
    w = w_ref[0] + w_ref[1]
    o_ref[...] = jnp.dot(h_ref[...], w, preferred_element_type=jnp.float32)


def _matmul(h, weight):
    grid = 10
    rows = N_NODES // grid
    return pl.pallas_call(
        _mm_body,
        grid=(grid,),
        in_specs=[
            pl.BlockSpec((rows, D), lambda i: (i, 0)),
            pl.BlockSpec((2, D, D), lambda i: (0, 0, 0)),
        ],
        out_specs=pl.BlockSpec((rows, D), lambda i: (i, 0)),
        out_shape=jax.ShapeDtypeStruct((N_NODES, D), jnp.float32),
    )(h, weight)


def _sm_body(x_ref, o_ref):
    x = x_ref[...]
    rid = lax.broadcasted_iota(jnp.int32, x.shape, 0)
    valid = rid < (N_EDGES // D)
    m = jnp.max(jnp.where(valid, x, -jnp.inf))
    ex = jnp.where(valid, jnp.exp(x - m), 0.0)
    o_ref[...] = ex / jnp.sum(ex)


def _softmax_padded(edge_feat):
    xp = jnp.pad(edge_feat, (0, E_PAD - N_EDGES)).reshape(E_PAD // D, D)
    out = pl.pallas_call(
        _sm_body,
        out_shape=jax.ShapeDtypeStruct((E_PAD // D, D), jnp.float32),
    )(xp)
    return out.reshape(E_PAD)


def _comb_body(p_ref, b_ref, o_ref):
    o_ref[...] = p_ref[0] + p_ref[1] + b_ref[...]


def _combine(partials, bias):
    grid = 10
    rows = N_NODES // grid
    return pl.pallas_call(
        _comb_body,
        grid=(grid,),
        in_specs=[
            pl.BlockSpec((2, rows, D), lambda i: (0, i, 0)),
            pl.BlockSpec((1, D), lambda i: (0, 0)),
        ],
        out_specs=pl.BlockSpec((rows, D), lambda i: (i, 0)),
        out_shape=jax.ShapeDtypeStruct((N_NODES, D), jnp.float32),
    )(partials, bias.reshape(1, D))


def _sc_body(hc_hbm, src_hbm, dst_hbm, e_hbm, out_hbm,
             acc, src_v, dst_v, e_v, rows_v, sem):
    cid = lax.axis_index("c")
    sid = lax.axis_index("s")
    wid = cid * NS + sid

    def _zero_row(i, _):
        for q in range(D // LANES):
            rows_v[i, pl.ds(q * LANES, LANES)] = jnp.zeros((LANES,), jnp.float32)
        return 0
    lax.fori_loop(0, CHUNK, _zero_row, 0)

    r0 = sid * ROWS_PER_TILE
    for k in range(4):
        pltpu.sync_copy(rows_v.at[pl.ds(0, 128)],
                        acc.at[pl.ds(r0 + k * 128, 128)])
    pltpu.sync_copy(rows_v.at[pl.ds(0, 112)], acc.at[pl.ds(r0 + 512, 112)])

    @pl.when(sid == NS - 1)
    def _zero_tail():
        pltpu.sync_copy(rows_v.at[pl.ds(0, 16)],
                        acc.at[pl.ds(NS * ROWS_PER_TILE, 16)])
    plsc.subcore_barrier()

    pltpu.sync_copy(src_hbm.at[wid], src_v)
    pltpu.sync_copy(dst_hbm.at[wid], dst_v)
    pltpu.sync_copy(e_hbm.at[wid], e_v)

    def _chunk(c, _):
        pltpu.async_copy(hc_hbm.at[src_v.at[c]], rows_v, sem).wait()

        def _group(g, _):
            ev = e_v[c, pl.ds(g * LANES, LANES)]
            for j in range(LANES):
                s = ev[j]
                row = g * LANES + j
                for q in range(D // LANES):
                    sl = pl.ds(q * LANES, LANES)
                    rows_v[row, sl] = rows_v[row, sl] * s
            return 0
        lax.fori_loop(0, CHUNK // LANES, _group, 0)

        pltpu.sync_copy(rows_v, acc.at[dst_v.at[c]], add=True)
        return 0

    lax.fori_loop(0, CPW, _chunk, 0)
    plsc.subcore_barrier()

    pltpu.sync_copy(acc.at[pl.ds(r0, ROWS_PER_TILE)],
                    out_hbm.at[cid, pl.ds(r0, ROWS_PER_TILE)])

    @pl.when(sid == NS - 1)
    def _write_tail():
        t0 = NS * ROWS_PER_TILE
        pltpu.sync_copy(acc.at[pl.ds(t0, N_NODES - NS * ROWS_PER_TILE)],
                        out_hbm.at[cid, pl.ds(t0, N_NODES - NS * ROWS_PER_TILE)])


_sc_scatter = pl.kernel(
    _sc_body,
    out_type=jax.ShapeDtypeStruct((NC, N_NODES, D), jnp.float32),
    mesh=plsc.VectorSubcoreMesh(core_axis_name="c", subcore_axis_name="s"),
    scratch_types=[
        pltpu.VMEM_SHARED((N_NODES, D), jnp.float32),
        pltpu.VMEM((CPW, CHUNK), jnp.int32),
        pltpu.VMEM((CPW, CHUNK), jnp.int32),
        pltpu.VMEM((CPW, CHUNK), jnp.float32),
        pltpu.VMEM((CHUNK, D), jnp.float32),
        pltpu.SemaphoreType.DMA,
    ],
)


def kernel(h, edge_index, edge_feat, weight, bias):
    hc = _matmul(h, weight)
    e = _softmax_padded(edge_feat)
    pad = E_PAD - N_EDGES
    src = jnp.pad(edge_index[0], (0, pad)).reshape(NW, CPW, CHUNK)
    dst = jnp.pad(edge_index[1], (0, pad)).reshape(NW, CPW, CHUNK)
    partials = _sc_scatter(hc, src, dst, e.reshape(NW, CPW, CHUNK))
    return _combine(partials, bias)


# speedup vs baseline: 13.1494x; 2.4033x over previous
import jax
import jax.numpy as jnp
from jax import lax
from jax.experimental import pallas as pl
from jax.experimental.pallas import tpu as pltpu
from jax.experimental.pallas import tpu_sc as plsc

N_NODES = 10000
N_EDGES = 320000
D = 128

NC = 2
NS = 16
NW = NC * NS
CHUNK = 80
CPW = 125
EPW = CPW * CHUNK
ROWS_PER_TILE = 624
LANES = 16


def _mm_body(h_ref, w_ref, o_ref):
    w = w_ref[0] + w_ref[1]
    o_ref[...] = jnp.dot(h_ref[...], w, preferred_element_type=jnp.float32)


def _matmul(h, weight):
    grid = 10
    rows = N_NODES // grid
    return pl.pallas_call(
        _mm_body,
        grid=(grid,),
        in_specs=[
            pl.BlockSpec((rows, D), lambda i: (i, 0)),
            pl.BlockSpec((2, D, D), lambda i: (0, 0, 0)),
        ],
        out_specs=pl.BlockSpec((rows, D), lambda i: (i, 0)),
        out_shape=jax.ShapeDtypeStruct((N_NODES, D), jnp.float32),
    )(h, weight)


def _sm_body(x_ref, o_ref):
    x = x_ref[...]
    ex = jnp.exp(x - jnp.max(x))
    o_ref[...] = ex / jnp.sum(ex)


def _softmax(edge_feat):
    xp = edge_feat.reshape(N_EDGES // D, D)
    out = pl.pallas_call(
        _sm_body,
        out_shape=jax.ShapeDtypeStruct((N_EDGES // D, D), jnp.float32),
    )(xp)
    return out.reshape(N_EDGES)


def _comb_body(p_ref, b_ref, o_ref):
    o_ref[...] = p_ref[0] + p_ref[1] + b_ref[...]


def _combine(partials, bias):
    grid = 10
    rows = N_NODES // grid
    return pl.pallas_call(
        _comb_body,
        grid=(grid,),
        in_specs=[
            pl.BlockSpec((2, rows, D), lambda i: (0, i, 0)),
            pl.BlockSpec((1, D), lambda i: (0, 0)),
        ],
        out_specs=pl.BlockSpec((rows, D), lambda i: (i, 0)),
        out_shape=jax.ShapeDtypeStruct((N_NODES, D), jnp.float32),
    )(partials, bias.reshape(1, D))


def _sc_body(hc_hbm, src_hbm, dst_hbm, e_hbm, out_hbm,
             acc, is0, is1, is2, is3, id0, id1, id2, id3,
             ie0, ie1, ie2, ie3, g0, g1, s0, s1,
             gsem0, gsem1, ssem0, ssem1, icsem0, icsem1):
    cid = lax.axis_index("c")
    sid = lax.axis_index("s")
    wid = cid * NS + sid
    gbuf = (g0, g1)
    sbuf = (s0, s1)
    gsem = (gsem0, gsem1)
    ssem = (ssem0, ssem1)
    icsem = (icsem0, icsem1)
    isb = (is0, is1, is2, is3)
    idb = (id0, id1, id2, id3)
    ieb = (ie0, ie1, ie2, ie3)

    def _zero_row(i, _):
        for q in range(D // LANES):
            g0[i, pl.ds(q * LANES, LANES)] = jnp.zeros((LANES,), jnp.float32)
        return 0
    lax.fori_loop(0, CHUNK, _zero_row, 0)

    r0 = sid * ROWS_PER_TILE
    for k in range(7):
        pltpu.sync_copy(g0.at[pl.ds(0, CHUNK)],
                        acc.at[pl.ds(r0 + k * CHUNK, CHUNK)])
    pltpu.sync_copy(g0.at[pl.ds(0, 64)], acc.at[pl.ds(r0 + 560, 64)])

    @pl.when(sid == NS - 1)
    def _zero_tail():
        pltpu.sync_copy(g0.at[pl.ds(0, 16)],
                        acc.at[pl.ds(NS * ROWS_PER_TILE, 16)])
    plsc.subcore_barrier()

    for t in range(2):
        pltpu.sync_copy(src_hbm.at[wid, t], isb[t])
        pltpu.sync_copy(dst_hbm.at[wid, t], idb[t])
        pltpu.sync_copy(e_hbm.at[wid, t], ieb[t])
    pltpu.async_copy(hc_hbm.at[is0], g0, gsem0)

    def _turn(t, k):
        b = k % 2

        @pl.when(t >= 2)
        def _drain_scatter():
            pltpu.make_async_copy(sbuf[b], acc.at[idb[(k - 2) % 4]],
                                  ssem[b]).wait()

        pltpu.make_async_copy(hc_hbm.at[isb[k]], gbuf[b], gsem[b]).wait()

        @pl.when(t + 2 < CPW)
        def _stage_idx():
            s2 = (k + 2) % 4
            pltpu.async_copy(src_hbm.at[wid, t + 2], isb[s2], icsem[b])
            pltpu.async_copy(dst_hbm.at[wid, t + 2], idb[s2], icsem[b])
            pltpu.async_copy(e_hbm.at[wid, t + 2], ieb[s2], icsem[b])

        @pl.when(jnp.logical_and(t >= 1, t + 1 < CPW))
        def _drain_idx():
            s1_ = (k + 1) % 4
            pltpu.make_async_copy(src_hbm.at[wid, t + 1], isb[s1_],
                                  icsem[1 - b]).wait()
            pltpu.make_async_copy(dst_hbm.at[wid, t + 1], idb[s1_],
                                  icsem[1 - b]).wait()
            pltpu.make_async_copy(e_hbm.at[wid, t + 1], ieb[s1_],
                                  icsem[1 - b]).wait()

        @pl.when(t + 1 < CPW)
        def _next_gather():
            pltpu.async_copy(hc_hbm.at[isb[(k + 1) % 4]], gbuf[1 - b],
                             gsem[1 - b])

        def _group(g, _):
            ev = ieb[k][pl.ds(g * LANES, LANES)]
            for j in range(LANES):
                s = ev[j]
                row = g * LANES + j
                for q in range(D // LANES):
                    sl = pl.ds(q * LANES, LANES)
                    sbuf[b][row, sl] = gbuf[b][row, sl] * s
            return 0
        lax.fori_loop(0, CHUNK // LANES, _group, 0)

        pltpu.async_copy(sbuf[b], acc.at[idb[k]], ssem[b], add=True)

    def _iter(i, _):
        for k in range(4):
            _turn(4 * i + k, k)
        return 0
    lax.fori_loop(0, CPW // 4, _iter, 0)
    _turn(jnp.int32(CPW - 1), (CPW - 1) % 4)

    pltpu.make_async_copy(s1, acc.at[idb[(CPW - 2) % 4]], ssem1).wait()
    pltpu.make_async_copy(s0, acc.at[idb[(CPW - 1) % 4]], ssem0).wait()
    plsc.subcore_barrier()

    pltpu.sync_copy(acc.at[pl.ds(r0, ROWS_PER_TILE)],
                    out_hbm.at[cid, pl.ds(r0, ROWS_PER_TILE)])

    @pl.when(sid == NS - 1)
    def _write_tail():
        t0 = NS * ROWS_PER_TILE
        pltpu.sync_copy(acc.at[pl.ds(t0, N_NODES - NS * ROWS_PER_TILE)],
                        out_hbm.at[cid, pl.ds(t0, N_NODES - NS * ROWS_PER_TILE)])


_sc_scatter = pl.kernel(
    _sc_body,
    out_type=jax.ShapeDtypeStruct((NC, N_NODES, D), jnp.float32),
    mesh=plsc.VectorSubcoreMesh(core_axis_name="c", subcore_axis_name="s"),
    scratch_types=[
        pltpu.VMEM_SHARED((N_NODES, D), jnp.float32),
        pltpu.VMEM((CHUNK,), jnp.int32),
        pltpu.VMEM((CHUNK,), jnp.int32),
        pltpu.VMEM((CHUNK,), jnp.int32),
        pltpu.VMEM((CHUNK,), jnp.int32),
        pltpu.VMEM((CHUNK,), jnp.int32),
        pltpu.VMEM((CHUNK,), jnp.int32),
        pltpu.VMEM((CHUNK,), jnp.int32),
        pltpu.VMEM((CHUNK,), jnp.int32),
        pltpu.VMEM((CHUNK,), jnp.float32),
        pltpu.VMEM((CHUNK,), jnp.float32),
        pltpu.VMEM((CHUNK,), jnp.float32),
        pltpu.VMEM((CHUNK,), jnp.float32),
        pltpu.VMEM((CHUNK, D), jnp.float32),
        pltpu.VMEM((CHUNK, D), jnp.float32),
        pltpu.VMEM((CHUNK, D), jnp.float32),
        pltpu.VMEM((CHUNK, D), jnp.float32),
        pltpu.SemaphoreType.DMA,
        pltpu.SemaphoreType.DMA,
        pltpu.SemaphoreType.DMA,
        pltpu.SemaphoreType.DMA,
        pltpu.SemaphoreType.DMA,
        pltpu.SemaphoreType.DMA,
    ],
)


def kernel(h, edge_index, edge_feat, weight, bias):
    hc = _matmul(h, weight)
    e = _softmax(edge_feat)
    src = edge_index[0].reshape(NW, CPW, CHUNK)
    dst = edge_index[1].reshape(NW, CPW, CHUNK)
    partials = _sc_scatter(hc, src, dst, e.reshape(NW, CPW, CHUNK))
    return _combine(partials, bias)


# speedup vs baseline: 15.4929x; 1.1782x over previous
import jax
import jax.numpy as jnp
from jax import lax
from jax.experimental import pallas as pl
from jax.experimental.pallas import tpu as pltpu
from jax.experimental.pallas import tpu_sc as plsc

N_NODES = 10000
N_EDGES = 320000
D = 128

NC = 2
NS = 16
NW = NC * NS
CHUNK = 80
CPW = 125
EPW = CPW * CHUNK
ROWS_PER_TILE = 624
LANES = 16


def _mm_sm_body(h_ref, w_ref, ef_ref, hc_ref, e_ref):
    w = w_ref[0] + w_ref[1]
    hc_ref[...] = jnp.dot(h_ref[...], w, preferred_element_type=jnp.float32)

    @pl.when(pl.program_id(0) == 0)
    def _softmax():
        x = ef_ref[...]
        ex = jnp.exp(x - jnp.max(x))
        e_ref[...] = ex / jnp.sum(ex)


def _matmul_softmax(h, weight, edge_feat):
    grid = 10
    rows = N_NODES // grid
    erows = N_EDGES // D
    return pl.pallas_call(
        _mm_sm_body,
        grid=(grid,),
        in_specs=[
            pl.BlockSpec((rows, D), lambda i: (i, 0)),
            pl.BlockSpec((2, D, D), lambda i: (0, 0, 0)),
            pl.BlockSpec((erows, D), lambda i: (0, 0)),
        ],
        out_specs=[
            pl.BlockSpec((rows, D), lambda i: (i, 0)),
            pl.BlockSpec((erows, D), lambda i: (0, 0)),
        ],
        out_shape=[
            jax.ShapeDtypeStruct((N_NODES, D), jnp.float32),
            jax.ShapeDtypeStruct((erows, D), jnp.float32),
        ],
    )(h, weight, edge_feat.reshape(erows, D))


def _comb_body(p_ref, b_ref, o_ref):
    o_ref[...] = p_ref[0] + p_ref[1] + b_ref[...]


def _combine(partials, bias):
    grid = 10
    rows = N_NODES // grid
    return pl.pallas_call(
        _comb_body,
        grid=(grid,),
        in_specs=[
            pl.BlockSpec((2, rows, D), lambda i: (0, i, 0)),
            pl.BlockSpec((1, D), lambda i: (0, 0)),
        ],
        out_specs=pl.BlockSpec((rows, D), lambda i: (i, 0)),
        out_shape=jax.ShapeDtypeStruct((N_NODES, D), jnp.float32),
    )(partials, bias.reshape(1, D))


def _sc_body(hc_hbm, src_hbm, dst_hbm, e_hbm, out_hbm,
             acc, is0, is1, is2, is3, id0, id1, id2, id3,
             ie0, ie1, ie2, ie3, g0, g1, s0, s1,
             gsem0, gsem1, ssem0, ssem1, icsem0, icsem1):
    cid = lax.axis_index("c")
    sid = lax.axis_index("s")
    wid = cid * NS + sid
    gbuf = (g0, g1)
    sbuf = (s0, s1)
    gsem = (gsem0, gsem1)
    ssem = (ssem0, ssem1)
    icsem = (icsem0, icsem1)
    isb = (is0, is1, is2, is3)
    idb = (id0, id1, id2, id3)
    ieb = (ie0, ie1, ie2, ie3)

    def _zero_row(i, _):
        for q in range(D // LANES):
            g0[i, pl.ds(q * LANES, LANES)] = jnp.zeros((LANES,), jnp.float32)
        return 0
    lax.fori_loop(0, CHUNK, _zero_row, 0)

    r0 = sid * ROWS_PER_TILE
    for k in range(7):
        pltpu.sync_copy(g0.at[pl.ds(0, CHUNK)],
                        acc.at[pl.ds(r0 + k * CHUNK, CHUNK)])
    pltpu.sync_copy(g0.at[pl.ds(0, 64)], acc.at[pl.ds(r0 + 560, 64)])

    @pl.when(sid == NS - 1)
    def _zero_tail():
        pltpu.sync_copy(g0.at[pl.ds(0, 16)],
                        acc.at[pl.ds(NS * ROWS_PER_TILE, 16)])
    plsc.subcore_barrier()

    for t in range(2):
        pltpu.sync_copy(src_hbm.at[wid, t], isb[t])
        pltpu.sync_copy(dst_hbm.at[wid, t], idb[t])
        pltpu.sync_copy(e_hbm.at[wid, t], ieb[t])
    pltpu.async_copy(hc_hbm.at[is0], g0, gsem0)

    def _turn(t, k):
        b = k % 2

        @pl.when(t >= 2)
        def _drain_scatter():
            pltpu.make_async_copy(sbuf[b], acc.at[idb[(k - 2) % 4]],
                                  ssem[b]).wait()

        pltpu.make_async_copy(hc_hbm.at[isb[k]], gbuf[b], gsem[b]).wait()

        @pl.when(t + 2 < CPW)
        def _stage_idx():
            s2 = (k + 2) % 4
            pltpu.async_copy(src_hbm.at[wid, t + 2], isb[s2], icsem[b])
            pltpu.async_copy(dst_hbm.at[wid, t + 2], idb[s2], icsem[b])
            pltpu.async_copy(e_hbm.at[wid, t + 2], ieb[s2], icsem[b])

        @pl.when(jnp.logical_and(t >= 1, t + 1 < CPW))
        def _drain_idx():
            s1_ = (k + 1) % 4
            pltpu.make_async_copy(src_hbm.at[wid, t + 1], isb[s1_],
                                  icsem[1 - b]).wait()
            pltpu.make_async_copy(dst_hbm.at[wid, t + 1], idb[s1_],
                                  icsem[1 - b]).wait()
            pltpu.make_async_copy(e_hbm.at[wid, t + 1], ieb[s1_],
                                  icsem[1 - b]).wait()

        @pl.when(t + 1 < CPW)
        def _next_gather():
            pltpu.async_copy(hc_hbm.at[isb[(k + 1) % 4]], gbuf[1 - b],
                             gsem[1 - b])

        @plsc.parallel_loop(0, CHUNK, step=LANES)
        def _group(g):
            ev = ieb[k][pl.ds(g, LANES)]
            for j in range(LANES):
                s = ev[j]
                row = g + j
                for q in range(D // LANES):
                    sl = pl.ds(q * LANES, LANES)
                    sbuf[b][row, sl] = gbuf[b][row, sl] * s

        pltpu.async_copy(sbuf[b], acc.at[idb[k]], ssem[b], add=True)

    def _iter(i, _):
        for k in range(4):
            _turn(4 * i + k, k)
        return 0
    lax.fori_loop(0, CPW // 4, _iter, 0)
    _turn(jnp.int32(CPW - 1), (CPW - 1) % 4)

    pltpu.make_async_copy(s1, acc.at[idb[(CPW - 2) % 4]], ssem1).wait()
    pltpu.make_async_copy(s0, acc.at[idb[(CPW - 1) % 4]], ssem0).wait()
    plsc.subcore_barrier()

    pltpu.sync_copy(acc.at[pl.ds(r0, ROWS_PER_TILE)],
                    out_hbm.at[cid, pl.ds(r0, ROWS_PER_TILE)])

    @pl.when(sid == NS - 1)
    def _write_tail():
        t0 = NS * ROWS_PER_TILE
        pltpu.sync_copy(acc.at[pl.ds(t0, N_NODES - NS * ROWS_PER_TILE)],
                        out_hbm.at[cid, pl.ds(t0, N_NODES - NS * ROWS_PER_TILE)])


_sc_scatter = pl.kernel(
    _sc_body,
    out_type=jax.ShapeDtypeStruct((NC, N_NODES, D), jnp.float32),
    mesh=plsc.VectorSubcoreMesh(core_axis_name="c", subcore_axis_name="s"),
    scratch_types=[
        pltpu.VMEM_SHARED((N_NODES, D), jnp.float32),
        pltpu.VMEM((CHUNK,), jnp.int32),
        pltpu.VMEM((CHUNK,), jnp.int32),
        pltpu.VMEM((CHUNK,), jnp.int32),
        pltpu.VMEM((CHUNK,), jnp.int32),
        pltpu.VMEM((CHUNK,), jnp.int32),
        pltpu.VMEM((CHUNK,), jnp.int32),
        pltpu.VMEM((CHUNK,), jnp.int32),
        pltpu.VMEM((CHUNK,), jnp.int32),
        pltpu.VMEM((CHUNK,), jnp.float32),
        pltpu.VMEM((CHUNK,), jnp.float32),
        pltpu.VMEM((CHUNK,), jnp.float32),
        pltpu.VMEM((CHUNK,), jnp.float32),
        pltpu.VMEM((CHUNK, D), jnp.float32),
        pltpu.VMEM((CHUNK, D), jnp.float32),
        pltpu.VMEM((CHUNK, D), jnp.float32),
        pltpu.VMEM((CHUNK, D), jnp.float32),
        pltpu.SemaphoreType.DMA,
        pltpu.SemaphoreType.DMA,
        pltpu.SemaphoreType.DMA,
        pltpu.SemaphoreType.DMA,
        pltpu.SemaphoreType.DMA,
        pltpu.SemaphoreType.DMA,
    ],
)


def kernel(h, edge_index, edge_feat, weight, bias):
    hc, e = _matmul_softmax(h, weight, edge_feat)
    src = edge_index[0].reshape(NW, CPW, CHUNK)
    dst = edge_index[1].reshape(NW, CPW, CHUNK)
    partials = _sc_scatter(hc, src, dst, e.reshape(NW, CPW, CHUNK))
    return _combine(partials, bias)
